# baseline (device time: 83132 ns/iter reference)
import jax
import jax.numpy as jnp
from jax import lax
from jax.experimental import pallas as pl
from jax.experimental.pallas import tpu as pltpu

N_DEV = 16
N_STAGES = 4
N_LAYERS = 3


def kernel(x, Win0, Wout0, Win1, Wout1, Win2, Wout2):
    b, d = x.shape

    def body(x_ref, win0_ref, wout0_ref, win1_ref, wout1_ref, win2_ref,
             wout2_ref, out_ref, send_ref, recv_ref, send_sems, recv_sems):
        my_i = lax.axis_index("i")

        wins = [win0_ref, win1_ref, win2_ref]
        wouts = [wout0_ref, wout1_ref, wout2_ref]

        acc = x_ref[...]
        for l in range(N_LAYERS):
            h = jnp.dot(
                acc.astype(jnp.bfloat16),
                wins[l][...].astype(jnp.bfloat16),
                preferred_element_type=jnp.float32,
            )
            h = jnp.maximum(h, 0.0)
            acc = jnp.dot(
                h.astype(jnp.bfloat16),
                wouts[l][...].astype(jnp.bfloat16),
                preferred_element_type=jnp.float32,
            )
            for s in range(N_STAGES):
                k = l * N_STAGES + s
                partner = my_i ^ (1 << s)
                send_ref[...] = acc
                rdma = pltpu.make_async_remote_copy(
                    src_ref=send_ref,
                    dst_ref=recv_ref.at[k],
                    send_sem=send_sems.at[k],
                    recv_sem=recv_sems.at[k],
                    device_id=(partner,),
                    device_id_type=pl.DeviceIdType.MESH,
                )
                rdma.start()
                rdma.wait()
                acc = acc + recv_ref[k]

        out_ref[...] = acc

    return pl.pallas_call(
        body,
        out_shape=jax.ShapeDtypeStruct((b, d), jnp.float32),
        in_specs=[pl.BlockSpec(memory_space=pltpu.VMEM)] * 7,
        out_specs=pl.BlockSpec(memory_space=pltpu.VMEM),
        scratch_shapes=[
            pltpu.VMEM((b, d), jnp.float32),
            pltpu.VMEM((N_LAYERS * N_STAGES, b, d), jnp.float32),
            pltpu.SemaphoreType.DMA((N_LAYERS * N_STAGES,)),
            pltpu.SemaphoreType.DMA((N_LAYERS * N_STAGES,)),
        ],
    )(x, Win0, Wout0, Win1, Wout1, Win2, Wout2)


# device time: 56039 ns/iter; 1.4835x vs baseline; 1.4835x over previous
import jax
import jax.numpy as jnp
from jax import lax
from jax.experimental import pallas as pl
from jax.experimental.pallas import tpu as pltpu

N_DEV = 16
N_STAGES = 4
N_LAYERS = 3


def kernel(x, Win0, Wout0, Win1, Wout1, Win2, Wout2):
    b, d = x.shape

    def body(x_ref, win0_ref, wout0_ref, win1_ref, wout1_ref, win2_ref,
             wout2_ref, out_ref, send_ref, recv_ref, send_sems, recv_sems):
        my_i = lax.axis_index("i")

        barrier_sem = pltpu.get_barrier_semaphore()
        for s in range(N_STAGES):
            pl.semaphore_signal(
                barrier_sem, inc=1,
                device_id=(my_i ^ (1 << s),),
                device_id_type=pl.DeviceIdType.MESH,
            )
        pl.semaphore_wait(barrier_sem, N_STAGES)

        wins = [win0_ref, win1_ref, win2_ref]
        wouts = [wout0_ref, wout1_ref, wout2_ref]

        rdmas = []
        acc = x_ref[...]
        for l in range(N_LAYERS):
            h = jnp.dot(
                acc.astype(jnp.bfloat16),
                wins[l][...].astype(jnp.bfloat16),
                preferred_element_type=jnp.float32,
            )
            h = jnp.maximum(h, 0.0)
            acc = jnp.dot(
                h.astype(jnp.bfloat16),
                wouts[l][...].astype(jnp.bfloat16),
                preferred_element_type=jnp.float32,
            )
            for s in range(N_STAGES):
                k = l * N_STAGES + s
                partner = my_i ^ (1 << s)
                if k >= 2:
                    rdmas[k - 2].wait_send()
                send_ref[k % 2] = acc.astype(jnp.bfloat16)
                rdma = pltpu.make_async_remote_copy(
                    src_ref=send_ref.at[k % 2],
                    dst_ref=recv_ref.at[k],
                    send_sem=send_sems.at[k],
                    recv_sem=recv_sems.at[k],
                    device_id=(partner,),
                    device_id_type=pl.DeviceIdType.MESH,
                )
                rdma.start()
                rdmas.append(rdma)
                rdma.wait_recv()
                acc = acc + recv_ref[k].astype(jnp.float32)

        out_ref[...] = acc
        rdmas[-2].wait_send()
        rdmas[-1].wait_send()

    return pl.pallas_call(
        body,
        out_shape=jax.ShapeDtypeStruct((b, d), jnp.float32),
        in_specs=[pl.BlockSpec(memory_space=pltpu.VMEM)] * 7,
        out_specs=pl.BlockSpec(memory_space=pltpu.VMEM),
        scratch_shapes=[
            pltpu.VMEM((2, b, d), jnp.bfloat16),
            pltpu.VMEM((N_LAYERS * N_STAGES, b, d), jnp.bfloat16),
            pltpu.SemaphoreType.DMA((N_LAYERS * N_STAGES,)),
            pltpu.SemaphoreType.DMA((N_LAYERS * N_STAGES,)),
        ],
        compiler_params=pltpu.CompilerParams(collective_id=0),
    )(x, Win0, Wout0, Win1, Wout1, Win2, Wout2)


# device time: 49552 ns/iter; 1.6777x vs baseline; 1.1309x over previous
import jax
import jax.numpy as jnp
from jax import lax
from jax.experimental import pallas as pl
from jax.experimental.pallas import tpu as pltpu

N_DEV = 16
N_STAGES = 4
N_LAYERS = 3
N_CHUNKS = 2
N_SLOTS = N_LAYERS * N_STAGES * N_CHUNKS


def kernel(x, Win0, Wout0, Win1, Wout1, Win2, Wout2):
    b, d = x.shape
    rows = b // N_CHUNKS

    def body(x_ref, win0_ref, wout0_ref, win1_ref, wout1_ref, win2_ref,
             wout2_ref, out_ref, send_ref, recv_ref, send_sems, recv_sems):
        my_i = lax.axis_index("i")

        barrier_sem = pltpu.get_barrier_semaphore()
        for s in range(N_STAGES):
            pl.semaphore_signal(
                barrier_sem, inc=1,
                device_id=(my_i ^ (1 << s),),
                device_id_type=pl.DeviceIdType.MESH,
            )
        pl.semaphore_wait(barrier_sem, N_STAGES)

        wins = [win0_ref, win1_ref, win2_ref]
        wouts = [wout0_ref, wout1_ref, wout2_ref]
        rdmas = {}

        def compute(rows_f32, l):
            h = jnp.dot(
                rows_f32.astype(jnp.bfloat16),
                wins[l][...].astype(jnp.bfloat16),
                preferred_element_type=jnp.float32,
            )
            h = jnp.maximum(h, 0.0)
            return jnp.dot(
                h.astype(jnp.bfloat16),
                wouts[l][...].astype(jnp.bfloat16),
                preferred_element_type=jnp.float32,
            )

        def issue(c, l, s, p):
            k = (l * N_STAGES + s) * N_CHUNKS + c
            send_ref[k] = p.astype(jnp.bfloat16)
            rdma = pltpu.make_async_remote_copy(
                src_ref=send_ref.at[k],
                dst_ref=recv_ref.at[k],
                send_sem=send_sems.at[k],
                recv_sem=recv_sems.at[k],
                device_id=(my_i ^ (1 << s),),
                device_id_type=pl.DeviceIdType.MESH,
            )
            rdma.start()
            rdmas[k] = rdma

        def wait_add(c, l, s, p):
            k = (l * N_STAGES + s) * N_CHUNKS + c
            rdmas[k].wait_recv()
            return p + recv_ref[k].astype(jnp.float32)

        pA = compute(x_ref[0:rows, :], 0)
        issue(0, 0, 0, pA)
        pB = compute(x_ref[rows:b, :], 0)
        issue(1, 0, 0, pB)
        for l in range(N_LAYERS):
            for s in range(N_STAGES - 1):
                pA = wait_add(0, l, s, pA)
                issue(0, l, s + 1, pA)
                pB = wait_add(1, l, s, pB)
                issue(1, l, s + 1, pB)
            pA = wait_add(0, l, N_STAGES - 1, pA)
            if l < N_LAYERS - 1:
                pA = compute(pA, l + 1)
                issue(0, l + 1, 0, pA)
            pB = wait_add(1, l, N_STAGES - 1, pB)
            if l < N_LAYERS - 1:
                pB = compute(pB, l + 1)
                issue(1, l + 1, 0, pB)

        out_ref[0:rows, :] = pA
        out_ref[rows:b, :] = pB
        for k in range(N_SLOTS):
            rdmas[k].wait_send()

    return pl.pallas_call(
        body,
        out_shape=jax.ShapeDtypeStruct((b, d), jnp.float32),
        in_specs=[pl.BlockSpec(memory_space=pltpu.VMEM)] * 7,
        out_specs=pl.BlockSpec(memory_space=pltpu.VMEM),
        scratch_shapes=[
            pltpu.VMEM((N_SLOTS, rows, d), jnp.bfloat16),
            pltpu.VMEM((N_SLOTS, rows, d), jnp.bfloat16),
            pltpu.SemaphoreType.DMA((N_SLOTS,)),
            pltpu.SemaphoreType.DMA((N_SLOTS,)),
        ],
        compiler_params=pltpu.CompilerParams(collective_id=0),
    )(x, Win0, Wout0, Win1, Wout1, Win2, Wout2)


# device time: 47283 ns/iter; 1.7582x vs baseline; 1.0480x over previous
import jax
import jax.numpy as jnp
from jax import lax
from jax.experimental import pallas as pl
from jax.experimental.pallas import tpu as pltpu

N_DEV = 16
N_STAGES = 4
N_LAYERS = 3
N_CHUNKS = 2
N_SLOTS = N_LAYERS * N_STAGES * N_CHUNKS

STAGE_MASKS = (1, 3, 4, 8)


def kernel(x, Win0, Wout0, Win1, Wout1, Win2, Wout2):
    b, d = x.shape
    rows = b // N_CHUNKS

    def body(x_ref, win0_ref, wout0_ref, win1_ref, wout1_ref, win2_ref,
             wout2_ref, out_ref, send_ref, recv_ref, send_sems, recv_sems):
        my_i = lax.axis_index("i")

        barrier_sem = pltpu.get_barrier_semaphore()
        for m in STAGE_MASKS:
            pl.semaphore_signal(
                barrier_sem, inc=1,
                device_id=(my_i ^ m,),
                device_id_type=pl.DeviceIdType.MESH,
            )
        pl.semaphore_wait(barrier_sem, N_STAGES)

        wins = [win0_ref, win1_ref, win2_ref]
        wouts = [wout0_ref, wout1_ref, wout2_ref]
        rdmas = {}

        def compute(rows_f32, l):
            h = jnp.dot(
                rows_f32.astype(jnp.bfloat16),
                wins[l][...].astype(jnp.bfloat16),
                preferred_element_type=jnp.float32,
            )
            h = jnp.maximum(h, 0.0)
            return jnp.dot(
                h.astype(jnp.bfloat16),
                wouts[l][...].astype(jnp.bfloat16),
                preferred_element_type=jnp.float32,
            )

        def issue(c, l, s, p):
            k = (l * N_STAGES + s) * N_CHUNKS + c
            send_ref[k] = p.astype(jnp.bfloat16)
            rdma = pltpu.make_async_remote_copy(
                src_ref=send_ref.at[k],
                dst_ref=recv_ref.at[k],
                send_sem=send_sems.at[k],
                recv_sem=recv_sems.at[k],
                device_id=(my_i ^ STAGE_MASKS[s],),
                device_id_type=pl.DeviceIdType.MESH,
            )
            rdma.start()
            rdmas[k] = rdma

        def wait_add(c, l, s, p):
            k = (l * N_STAGES + s) * N_CHUNKS + c
            rdmas[k].wait_recv()
            return p + recv_ref[k].astype(jnp.float32)

        pA = compute(x_ref[0:rows, :], 0)
        issue(0, 0, 0, pA)
        pB = compute(x_ref[rows:b, :], 0)
        issue(1, 0, 0, pB)
        for l in range(N_LAYERS):
            for s in range(N_STAGES - 1):
                pA = wait_add(0, l, s, pA)
                issue(0, l, s + 1, pA)
                pB = wait_add(1, l, s, pB)
                issue(1, l, s + 1, pB)
            pA = wait_add(0, l, N_STAGES - 1, pA)
            if l < N_LAYERS - 1:
                pA = compute(pA, l + 1)
                issue(0, l + 1, 0, pA)
            pB = wait_add(1, l, N_STAGES - 1, pB)
            if l < N_LAYERS - 1:
                pB = compute(pB, l + 1)
                issue(1, l + 1, 0, pB)

        out_ref[0:rows, :] = pA
        out_ref[rows:b, :] = pB
        for k in range(N_SLOTS):
            rdmas[k].wait_send()

    return pl.pallas_call(
        body,
        out_shape=jax.ShapeDtypeStruct((b, d), jnp.float32),
        in_specs=[pl.BlockSpec(memory_space=pltpu.VMEM)] * 7,
        out_specs=pl.BlockSpec(memory_space=pltpu.VMEM),
        scratch_shapes=[
            pltpu.VMEM((N_SLOTS, rows, d), jnp.bfloat16),
            pltpu.VMEM((N_SLOTS, rows, d), jnp.bfloat16),
            pltpu.SemaphoreType.DMA((N_SLOTS,)),
            pltpu.SemaphoreType.DMA((N_SLOTS,)),
        ],
        compiler_params=pltpu.CompilerParams(collective_id=0),
    )(x, Win0, Wout0, Win1, Wout1, Win2, Wout2)


# device time: 44909 ns/iter; 1.8511x vs baseline; 1.0529x over previous
import jax
import jax.numpy as jnp
from jax import lax
from jax.experimental import pallas as pl
from jax.experimental.pallas import tpu as pltpu

N_DEV = 16
N_STAGES = 4
N_LAYERS = 3
N_CHUNKS = 2
N_SLOTS = N_LAYERS * N_STAGES * N_CHUNKS

STAGE_MASKS = ((1, 3, 4, 8), (4, 8, 1, 3))


def kernel(x, Win0, Wout0, Win1, Wout1, Win2, Wout2):
    b, d = x.shape
    rows = b // N_CHUNKS

    def body(x_ref, win0_ref, wout0_ref, win1_ref, wout1_ref, win2_ref,
             wout2_ref, out_ref, send_ref, recv_ref, send_sems, recv_sems):
        my_i = lax.axis_index("i")

        barrier_sem = pltpu.get_barrier_semaphore()
        for m in STAGE_MASKS[0]:
            pl.semaphore_signal(
                barrier_sem, inc=1,
                device_id=(my_i ^ m,),
                device_id_type=pl.DeviceIdType.MESH,
            )
        pl.semaphore_wait(barrier_sem, N_STAGES)

        wins = [win0_ref, win1_ref, win2_ref]
        wouts = [wout0_ref, wout1_ref, wout2_ref]
        rdmas = {}

        def compute(rows_f32, l):
            h = jnp.dot(
                rows_f32.astype(jnp.bfloat16),
                wins[l][...].astype(jnp.bfloat16),
                preferred_element_type=jnp.float32,
            )
            h = jnp.maximum(h, 0.0)
            return jnp.dot(
                h.astype(jnp.bfloat16),
                wouts[l][...].astype(jnp.bfloat16),
                preferred_element_type=jnp.float32,
            )

        def issue(c, l, s, p):
            k = (l * N_STAGES + s) * N_CHUNKS + c
            send_ref[k] = p.astype(jnp.bfloat16)
            rdma = pltpu.make_async_remote_copy(
                src_ref=send_ref.at[k],
                dst_ref=recv_ref.at[k],
                send_sem=send_sems.at[k],
                recv_sem=recv_sems.at[k],
                device_id=(my_i ^ STAGE_MASKS[c][s],),
                device_id_type=pl.DeviceIdType.MESH,
            )
            rdma.start()
            rdmas[k] = rdma

        def wait_add(c, l, s, p):
            k = (l * N_STAGES + s) * N_CHUNKS + c
            rdmas[k].wait_recv()
            return p + recv_ref[k].astype(jnp.float32)

        pA = compute(x_ref[0:rows, :], 0)
        issue(0, 0, 0, pA)
        pB = compute(x_ref[rows:b, :], 0)
        issue(1, 0, 0, pB)
        for l in range(N_LAYERS):
            for s in range(N_STAGES - 1):
                pA = wait_add(0, l, s, pA)
                issue(0, l, s + 1, pA)
                pB = wait_add(1, l, s, pB)
                issue(1, l, s + 1, pB)
            pA = wait_add(0, l, N_STAGES - 1, pA)
            if l < N_LAYERS - 1:
                pA = compute(pA, l + 1)
                issue(0, l + 1, 0, pA)
            pB = wait_add(1, l, N_STAGES - 1, pB)
            if l < N_LAYERS - 1:
                pB = compute(pB, l + 1)
                issue(1, l + 1, 0, pB)

        out_ref[0:rows, :] = pA
        out_ref[rows:b, :] = pB
        for k in range(N_SLOTS):
            rdmas[k].wait_send()

    return pl.pallas_call(
        body,
        out_shape=jax.ShapeDtypeStruct((b, d), jnp.float32),
        in_specs=[pl.BlockSpec(memory_space=pltpu.VMEM)] * 7,
        out_specs=pl.BlockSpec(memory_space=pltpu.VMEM),
        scratch_shapes=[
            pltpu.VMEM((N_SLOTS, rows, d), jnp.bfloat16),
            pltpu.VMEM((N_SLOTS, rows, d), jnp.bfloat16),
            pltpu.SemaphoreType.DMA((N_SLOTS,)),
            pltpu.SemaphoreType.DMA((N_SLOTS,)),
        ],
        compiler_params=pltpu.CompilerParams(collective_id=0),
    )(x, Win0, Wout0, Win1, Wout1, Win2, Wout2)
